# Initial kernel scaffold; baseline (speedup 1.0000x reference)
import jax
import jax.numpy as jnp
from jax.experimental import pallas as pl


def kernel(q, pos_codes, neg_codes, codebooks):
    raise NotImplementedError("write your pallas kernel here")


# speedup vs baseline: 129.1891x; 129.1891x over previous
import dataclasses
import functools

import jax
import jax.numpy as jnp
from jax import lax
from jax.experimental import pallas as pl
from jax.experimental.pallas import tpu as pltpu
from jax.experimental.pallas import tpu_sc as plsc

B = 16384
M = 96
K = 256
DSUB = 8
D = M * DSUB

NC = 2
NS = 16
L = 16
NW = NC * NS
ROWS_PER_W = B // NW
RBLK = 16
NBLK = ROWS_PER_W // RBLK
IDX_PER_BLK = RBLK * M
GCHUNK = 128
NGC = IDX_PER_BLK // GCHUNK
JCH = D // L


def _sc_diff_partials(q, cp, cn, table):
    mesh = plsc.VectorSubcoreMesh(core_axis_name="c", subcore_axis_name="s")

    cparams = pltpu.CompilerParams()
    for _field, _val in (("needs_layout_passes", False),
                         ("use_tc_tiling_on_sc", False)):
        if _field in pltpu.CompilerParams.__dataclass_fields__:
            cparams = dataclasses.replace(cparams, **{_field: _val})

    @functools.partial(
        pl.kernel,
        out_type=jax.ShapeDtypeStruct((B, L), jnp.float32),
        mesh=mesh,
        compiler_params=cparams,
        scratch_types=[
            pltpu.VMEM((RBLK, M), jnp.int32),
            pltpu.VMEM((RBLK, M), jnp.int32),
            pltpu.VMEM((IDX_PER_BLK,), jnp.int32),
            pltpu.VMEM((IDX_PER_BLK,), jnp.int32),
            pltpu.VMEM((RBLK, D), jnp.float32),
            pltpu.VMEM((IDX_PER_BLK, DSUB), jnp.float32),
            pltpu.VMEM((IDX_PER_BLK, DSUB), jnp.float32),
            pltpu.VMEM((RBLK, L), jnp.float32),
            pltpu.SemaphoreType.DMA,
            pltpu.SemaphoreType.DMA,
            pltpu.SemaphoreType.DMA,
        ],
    )
    def sc_kernel(q_hbm, cp_hbm, cn_hbm, tab_hbm, out_hbm,
                  cpv, cnv, ixp, ixn, qv, ebp, ebn, dacc,
                  sem_c, sem_g, sem_q):
        wid = lax.axis_index("c") * NS + lax.axis_index("s")
        base = wid * ROWS_PER_W

        lane = lax.iota(jnp.int32, L)
        lane_off = lane * K
        rpat = lax.shift_right_logical(lane, 3)
        cpat = lax.bitwise_and(lane, 7)

        @pl.loop(0, NBLK)
        def _blk(blk):
            row0 = base + blk * RBLK
            ccp = pltpu.async_copy(cp_hbm.at[pl.ds(row0, RBLK)], cpv, sem_c)
            ccn = pltpu.async_copy(cn_hbm.at[pl.ds(row0, RBLK)], cnv, sem_c)
            cq = pltpu.async_copy(q_hbm.at[pl.ds(row0, RBLK)], qv, sem_q)
            ccp.wait()
            ccn.wait()

            @pl.loop(0, RBLK)
            def _r(r):
                rb = r * M
                for c in range(M // L):
                    offs = lane_off + c * (L * K)
                    ixp[pl.ds(rb + c * L, L)] = cpv[r, pl.ds(c * L, L)] + offs
                    ixn[pl.ds(rb + c * L, L)] = cnv[r, pl.ds(c * L, L)] + offs

            gathers = []
            for g in range(NGC):
                sl = pl.ds(g * GCHUNK, GCHUNK)
                gathers.append(pltpu.async_copy(
                    tab_hbm.at[ixp.at[sl]], ebp.at[sl], sem_g))
                gathers.append(pltpu.async_copy(
                    tab_hbm.at[ixn.at[sl]], ebn.at[sl], sem_g))
            for g in gathers:
                g.wait()
            cq.wait()

            @pl.loop(0, RBLK)
            def _row(r):
                rb = r * M
                acc = jnp.zeros((L,), jnp.float32)
                for j in range(JCH):
                    qreg = qv[r, pl.ds(j * L, L)]
                    rp = rpat + (rb + 2 * j)
                    ep = plsc.load_gather(ebp, [rp, cpat])
                    en = plsc.load_gather(ebn, [rp, cpat])
                    acc = acc + qreg * (en - ep)
                dacc[r, :] = acc

            pltpu.sync_copy(dacc, out_hbm.at[pl.ds(row0, RBLK)])

    return sc_kernel(q, cp, cn, table)


def _tc_loss(dparts):
    def body(x_ref, o_ref):
        d = jnp.sum(x_ref[...], axis=1)
        sp = jnp.maximum(d, 0.0) + jnp.log1p(jnp.exp(-jnp.abs(d)))
        o_ref[...] = jnp.reshape(jnp.sum(sp) * (1.0 / B), (1, 1))

    out = pl.pallas_call(
        body,
        out_shape=jax.ShapeDtypeStruct((1, 1), jnp.float32),
    )(dparts)
    return out[0, 0]


def kernel(q, pos_codes, neg_codes, codebooks):
    table = codebooks.reshape(M * K, DSUB)
    cp = pos_codes.astype(jnp.int32)
    cn = neg_codes.astype(jnp.int32)
    dparts = _sc_diff_partials(q, cp, cn, table)
    return _tc_loss(dparts)


# speedup vs baseline: 164.7949x; 1.2756x over previous
import dataclasses
import functools

import jax
import jax.numpy as jnp
from jax import lax
from jax.experimental import pallas as pl
from jax.experimental.pallas import tpu as pltpu
from jax.experimental.pallas import tpu_sc as plsc

B = 16384
M = 96
K = 256
DSUB = 8
D = M * DSUB

NC = 2
NS = 16
L = 16
NW = NC * NS
ROWS_PER_W = B // NW
RBLK = 16
NBLK = ROWS_PER_W // RBLK
IDX_PER_BLK = RBLK * M
GCHUNK = 128
NGC = IDX_PER_BLK // GCHUNK
JCH = D // L


def _sc_diff_partials(q, cp, cn, table):
    mesh = plsc.VectorSubcoreMesh(core_axis_name="c", subcore_axis_name="s")

    cparams = pltpu.CompilerParams()
    for _field, _val in (("needs_layout_passes", False),
                         ("use_tc_tiling_on_sc", False)):
        if _field in pltpu.CompilerParams.__dataclass_fields__:
            cparams = dataclasses.replace(cparams, **{_field: _val})

    @functools.partial(
        pl.kernel,
        out_type=jax.ShapeDtypeStruct((B, L), jnp.float32),
        mesh=mesh,
        compiler_params=cparams,
        scratch_types=[
            pltpu.VMEM((2, RBLK, M), jnp.int32),
            pltpu.VMEM((2, RBLK, M), jnp.int32),
            pltpu.VMEM((2, IDX_PER_BLK), jnp.int32),
            pltpu.VMEM((2, IDX_PER_BLK), jnp.int32),
            pltpu.VMEM((2, RBLK, D), jnp.float32),
            pltpu.VMEM((2, IDX_PER_BLK, DSUB), jnp.float32),
            pltpu.VMEM((2, IDX_PER_BLK, DSUB), jnp.float32),
            pltpu.VMEM((RBLK, L), jnp.float32),
            pltpu.SemaphoreType.DMA,
            pltpu.SemaphoreType.DMA,
            pltpu.SemaphoreType.DMA,
            pltpu.SemaphoreType.DMA,
            pltpu.SemaphoreType.DMA,
            pltpu.SemaphoreType.DMA,
        ],
    )
    def sc_kernel(q_hbm, cp_hbm, cn_hbm, tab_hbm, out_hbm,
                  cpv, cnv, ixp, ixn, qv, ebp, ebn, dacc,
                  sem_c0, sem_c1, sem_g0, sem_g1, sem_q0, sem_q1):
        sems_c = (sem_c0, sem_c1)
        sems_g = (sem_g0, sem_g1)
        sems_q = (sem_q0, sem_q1)
        wid = lax.axis_index("c") * NS + lax.axis_index("s")
        base = wid * ROWS_PER_W

        lane = lax.iota(jnp.int32, L)
        lane_off = lane * K
        rpat = lax.shift_right_logical(lane, 3)
        cpat = lax.bitwise_and(lane, 7)

        def fire(blk, p):
            row0 = base + blk * RBLK
            c1 = pltpu.async_copy(cp_hbm.at[pl.ds(row0, RBLK)], cpv.at[p],
                                  sems_c[p])
            c2 = pltpu.async_copy(cn_hbm.at[pl.ds(row0, RBLK)], cnv.at[p],
                                  sems_c[p])
            pltpu.async_copy(q_hbm.at[pl.ds(row0, RBLK)], qv.at[p], sems_q[p])
            c1.wait()
            c2.wait()

            @pl.loop(0, RBLK)
            def _r(r):
                rb = r * M
                for c in range(M // L):
                    offs = lane_off + c * (L * K)
                    ixp.at[p][pl.ds(rb + c * L, L)] = (
                        cpv.at[p][r, pl.ds(c * L, L)] + offs)
                    ixn.at[p][pl.ds(rb + c * L, L)] = (
                        cnv.at[p][r, pl.ds(c * L, L)] + offs)

            for g in range(NGC):
                sl = pl.ds(g * GCHUNK, GCHUNK)
                pltpu.async_copy(tab_hbm.at[ixp.at[p].at[sl]],
                                 ebp.at[p].at[sl], sems_g[p])
                pltpu.async_copy(tab_hbm.at[ixn.at[p].at[sl]],
                                 ebn.at[p].at[sl], sems_g[p])

        def drain(p):
            pltpu.make_async_copy(q_hbm.at[pl.ds(0, RBLK)], qv.at[p],
                                  sems_q[p]).wait()
            pltpu.make_async_copy(tab_hbm.at[pl.ds(0, IDX_PER_BLK)],
                                  ebp.at[p], sems_g[p]).wait()
            pltpu.make_async_copy(tab_hbm.at[pl.ds(0, IDX_PER_BLK)],
                                  ebn.at[p], sems_g[p]).wait()

        def compute(blk, p):
            row0 = base + blk * RBLK

            @pl.loop(0, RBLK)
            def _row(r):
                rb = r * M
                acc = jnp.zeros((L,), jnp.float32)
                for j in range(JCH):
                    qreg = qv.at[p][r, pl.ds(j * L, L)]
                    rp = rpat + (rb + 2 * j)
                    ep = plsc.load_gather(ebp.at[p], [rp, cpat])
                    en = plsc.load_gather(ebn.at[p], [rp, cpat])
                    acc = acc + qreg * (en - ep)
                dacc[r, :] = acc

            pltpu.sync_copy(dacc, out_hbm.at[pl.ds(row0, RBLK)])

        fire(0, 0)

        @pl.loop(0, NBLK - 2, step=2)
        def _pair(blk0):
            for pp in (0, 1):
                blk = blk0 + pp
                fire(blk + 1, 1 - pp)
                drain(pp)
                compute(blk, pp)

        fire(NBLK - 1, 1)
        drain(0)
        compute(NBLK - 2, 0)
        drain(1)
        compute(NBLK - 1, 1)

    return sc_kernel(q, cp, cn, table)


def _tc_loss(dparts):
    def body(x_ref, o_ref):
        d = jnp.sum(x_ref[...], axis=1)
        sp = jnp.maximum(d, 0.0) + jnp.log1p(jnp.exp(-jnp.abs(d)))
        o_ref[...] = jnp.reshape(jnp.sum(sp) * (1.0 / B), (1, 1))

    out = pl.pallas_call(
        body,
        out_shape=jax.ShapeDtypeStruct((1, 1), jnp.float32),
    )(dparts)
    return out[0, 0]


def kernel(q, pos_codes, neg_codes, codebooks):
    table = codebooks.reshape(M * K, DSUB)
    cp = pos_codes.astype(jnp.int32)
    cn = neg_codes.astype(jnp.int32)
    dparts = _sc_diff_partials(q, cp, cn, table)
    return _tc_loss(dparts)
